# baseline (speedup 1.0000x reference)
import functools

import jax
import jax.numpy as jnp
from jax import lax
from jax.experimental import pallas as pl
from jax.experimental.pallas import tpu as pltpu
from jax.experimental.pallas import tpu_sc as plsc

Q = 0.7
K = 0.5
B = 4096
C = 1000
LQK = (1.0 - K ** Q) / Q

BR = 512
NBLK = B // BR

NUM_SC_CORES = 1


def _make_weight_gather():
    info = plsc.get_sparse_core_info()
    NC, NS, L = NUM_SC_CORES, info.num_subcores, info.num_lanes
    NW = NC * NS
    per_w = B // NW
    chunks = per_w // L

    mesh = plsc.VectorSubcoreMesh(
        core_axis_name="c", subcore_axis_name="s", num_cores=NC)

    @functools.partial(
        pl.kernel,
        mesh=mesh,
        out_type=jax.ShapeDtypeStruct((NW, L), jnp.float32),
        scratch_types=[
            pltpu.VMEM((per_w,), jnp.int32),
            pltpu.VMEM((per_w,), jnp.float32),
            pltpu.VMEM((L,), jnp.float32),
            pltpu.SemaphoreType.DMA,
        ],
    )
    def wgather(idx_hbm, table_hbm, out_hbm, idx_v, rows_v, acc_v, sem):
        wid = lax.axis_index("s") * NC + lax.axis_index("c")
        base = wid * per_w
        pltpu.sync_copy(idx_hbm.at[pl.ds(base, per_w)], idx_v)
        pltpu.async_copy(table_hbm.at[idx_v], rows_v, sem).wait()
        acc = rows_v[pl.ds(0, L)]
        for cidx in range(1, chunks):
            acc = acc + rows_v[pl.ds(cidx * L, L)]
        acc_v[...] = acc
        pltpu.sync_copy(acc_v, out_hbm.at[wid])

    return wgather, NW, L


def _tc_body(x_hbm, t_ref, o_ref, buf, sems):
    def block_copy(i, slot):
        return pltpu.make_async_copy(
            x_hbm.at[pl.ds(i * BR, BR), :], buf.at[slot], sems.at[slot])

    block_copy(0, 0).start()
    acc = jnp.zeros((1, 1), jnp.float32)
    for i in range(NBLK):
        slot = i % 2
        if i + 1 < NBLK:
            block_copy(i + 1, 1 - slot).start()
        block_copy(i, slot).wait()
        x = buf[slot]
        t = t_ref[pl.ds(i * BR, BR), :]
        col = lax.broadcasted_iota(jnp.int32, x.shape, 1)
        rowmax = jnp.max(x, axis=1, keepdims=True)
        sumexp = jnp.sum(jnp.exp(x - rowmax), axis=1, keepdims=True)
        tl = jnp.max(jnp.where(col == t, x, -jnp.inf), axis=1, keepdims=True)
        logp = tl - rowmax - jnp.log(sumexp)
        lq = (1.0 - jnp.exp(Q * logp)) * (1.0 / Q)
        acc = acc + jnp.sum(lq, axis=0, keepdims=True)
    o_ref[...] = acc


def kernel(logits, targets, indexes, weight):
    wgather, NW, L = _make_weight_gather()
    w_parts = wgather(indexes.astype(jnp.int32), weight.reshape(-1))

    t2d = targets.astype(jnp.int32).reshape(B, 1)
    logits = pltpu.with_memory_space_constraint(logits, pltpu.MemorySpace.HBM)
    lqsum = pl.pallas_call(
        _tc_body,
        in_specs=[
            pl.BlockSpec(memory_space=pltpu.MemorySpace.HBM),
            pl.BlockSpec((B, 1), lambda: (0, 0)),
        ],
        out_specs=pl.BlockSpec((1, 1), lambda: (0, 0)),
        out_shape=jax.ShapeDtypeStruct((1, 1), jnp.float32),
        scratch_shapes=[
            pltpu.VMEM((2, BR, C), jnp.float32),
            pltpu.SemaphoreType.DMA((2,)),
        ],
    )(logits, t2d)
    return (lqsum[0, 0] * (1.0 / B) - LQK) * (jnp.sum(w_parts) * (1.0 / B))


# speedup vs baseline: 1.6629x; 1.6629x over previous
import functools

import jax
import jax.numpy as jnp
from jax import lax
from jax.experimental import pallas as pl
from jax.experimental.pallas import tpu as pltpu
from jax.experimental.pallas import tpu_sc as plsc

Q = 0.7
K = 0.5
B = 4096
C = 1000
LQK = (1.0 - K ** Q) / Q

BC = 512
NBLK = B // BC

NUM_SC_CORES = 1


def _make_weight_gather():
    info = plsc.get_sparse_core_info()
    NC, NS, L = NUM_SC_CORES, info.num_subcores, info.num_lanes
    NW = NC * NS
    per_w = B // NW
    chunks = per_w // L

    mesh = plsc.VectorSubcoreMesh(
        core_axis_name="c", subcore_axis_name="s", num_cores=NC)

    @functools.partial(
        pl.kernel,
        mesh=mesh,
        out_type=jax.ShapeDtypeStruct((NW, L), jnp.float32),
        scratch_types=[
            pltpu.VMEM((per_w,), jnp.int32),
            pltpu.VMEM((per_w,), jnp.float32),
            pltpu.VMEM((L,), jnp.float32),
            pltpu.SemaphoreType.DMA,
        ],
    )
    def wgather(idx_hbm, table_hbm, out_hbm, idx_v, rows_v, acc_v, sem):
        wid = lax.axis_index("s") * NC + lax.axis_index("c")
        base = wid * per_w
        pltpu.sync_copy(idx_hbm.at[pl.ds(base, per_w)], idx_v)
        pltpu.async_copy(table_hbm.at[idx_v], rows_v, sem).wait()
        acc = rows_v[pl.ds(0, L)]
        for cidx in range(1, chunks):
            acc = acc + rows_v[pl.ds(cidx * L, L)]
        acc_v[...] = acc
        pltpu.sync_copy(acc_v, out_hbm.at[wid])

    return wgather, NW, L


def _tc_body(x_ref, t_ref, o_ref):
    i = pl.program_id(0)
    x = x_ref[...]
    t = t_ref[...].reshape(1, BC)
    row = lax.broadcasted_iota(jnp.int32, x.shape, 0)
    colmax = jnp.max(x, axis=0, keepdims=True)
    sumexp = jnp.sum(jnp.exp(x - colmax), axis=0, keepdims=True)
    tl = jnp.max(jnp.where(row == t, x, -jnp.inf), axis=0, keepdims=True)
    logp = tl - colmax - jnp.log(sumexp)
    lq = (1.0 - jnp.exp(Q * logp)) * (1.0 / Q)
    part = jnp.sum(lq, axis=1, keepdims=True)

    @pl.when(i == 0)
    def _init():
        o_ref[...] = jnp.zeros_like(part)

    o_ref[...] += part


def kernel(logits, targets, indexes, weight):
    wgather, NW, L = _make_weight_gather()
    w_parts = wgather(indexes.astype(jnp.int32), weight.reshape(-1))

    xt = logits.T
    xt = pltpu.with_memory_space_constraint(xt, pltpu.MemorySpace.HBM)
    t3 = targets.astype(jnp.int32).reshape(NBLK, 1, BC)
    lqsum = pl.pallas_call(
        _tc_body,
        grid=(NBLK,),
        in_specs=[
            pl.BlockSpec((C, BC), lambda i: (0, i)),
            pl.BlockSpec((1, 1, BC), lambda i: (i, 0, 0)),
        ],
        out_specs=pl.BlockSpec((1, 1), lambda i: (0, 0)),
        out_shape=jax.ShapeDtypeStruct((1, 1), jnp.float32),
    )(xt, t3)
    return (lqsum[0, 0] * (1.0 / B) - LQK) * (jnp.sum(w_parts) * (1.0 / B))


# speedup vs baseline: 1.6714x; 1.0051x over previous
import functools

import jax
import jax.numpy as jnp
from jax import lax
from jax.experimental import pallas as pl
from jax.experimental.pallas import tpu as pltpu
from jax.experimental.pallas import tpu_sc as plsc

Q = 0.7
K = 0.5
B = 4096
C = 1000
LQK = (1.0 - K ** Q) / Q

BC = 512
NBLK = B // BC

NUM_SC_CORES = 1


def _make_weight_gather():
    info = plsc.get_sparse_core_info()
    NC, NS, L = NUM_SC_CORES, info.num_subcores, info.num_lanes
    NW = NC * NS
    per_w = B // NW
    chunks = per_w // L

    mesh = plsc.VectorSubcoreMesh(
        core_axis_name="c", subcore_axis_name="s", num_cores=NC)

    @functools.partial(
        pl.kernel,
        mesh=mesh,
        out_type=jax.ShapeDtypeStruct((NW, L), jnp.float32),
        scratch_types=[
            pltpu.VMEM((per_w,), jnp.int32),
            pltpu.VMEM((per_w,), jnp.float32),
            pltpu.VMEM((L,), jnp.float32),
            pltpu.SemaphoreType.DMA,
        ],
    )
    def wgather(idx_hbm, table_hbm, out_hbm, idx_v, rows_v, acc_v, sem):
        wid = lax.axis_index("s") * NC + lax.axis_index("c")
        base = wid * per_w
        pltpu.sync_copy(idx_hbm.at[pl.ds(base, per_w)], idx_v)
        pltpu.async_copy(table_hbm.at[idx_v], rows_v, sem).wait()
        acc = rows_v[pl.ds(0, L)]
        for cidx in range(1, chunks):
            acc = acc + rows_v[pl.ds(cidx * L, L)]
        acc_v[...] = acc
        pltpu.sync_copy(acc_v, out_hbm.at[wid])

    return wgather, NW, L


def _tc_body(x_ref, t_ref, o_ref):
    i = pl.program_id(0)
    x = x_ref[...]
    t = t_ref[...].reshape(1, BC)
    row = lax.broadcasted_iota(jnp.int32, x.shape, 0)
    colmax = jnp.max(x, axis=0, keepdims=True)
    sumexp = jnp.sum(jnp.exp(x - colmax), axis=0, keepdims=True)
    tl = jnp.max(jnp.where(row == t, x, -jnp.inf), axis=0, keepdims=True)
    logp = tl - colmax - jnp.log(sumexp)
    lq = (1.0 - jnp.exp(Q * logp)) * (1.0 / Q)
    part = jnp.sum(lq, axis=1, keepdims=True)

    @pl.when(i == 0)
    def _init():
        o_ref[...] = jnp.zeros_like(part)

    o_ref[...] += part


def kernel(logits, targets, indexes, weight):
    wgather, NW, L = _make_weight_gather()
    w_parts = wgather(indexes.astype(jnp.int32), weight[:, 0])

    xt = logits.T
    xt = pltpu.with_memory_space_constraint(xt, pltpu.MemorySpace.HBM)
    t3 = targets.astype(jnp.int32).reshape(NBLK, 1, BC)
    lqsum = pl.pallas_call(
        _tc_body,
        grid=(NBLK,),
        in_specs=[
            pl.BlockSpec((C, BC), lambda i: (0, i)),
            pl.BlockSpec((1, 1, BC), lambda i: (i, 0, 0)),
        ],
        out_specs=pl.BlockSpec((1, 1), lambda i: (0, 0)),
        out_shape=jax.ShapeDtypeStruct((1, 1), jnp.float32),
    )(xt, t3)
    return (lqsum[0, 0] * (1.0 / B) - LQK) * (jnp.sum(w_parts) * (1.0 / B))
